# Initial kernel scaffold; baseline (speedup 1.0000x reference)
import jax
import jax.numpy as jnp
from jax.experimental import pallas as pl


def kernel(x, edge_index0, edge_index1, W0, W1, b):
    raise NotImplementedError("write your pallas kernel here")


# speedup vs baseline: 12.6411x; 12.6411x over previous
import jax
import jax.numpy as jnp
from jax import lax
from jax.experimental import pallas as pl
from jax.experimental.pallas import tpu as pltpu
from jax.experimental.pallas import tpu_sc as plsc

N = 50000
E = 800000
H = 32
NEG_SLOPE = 0.05

NC = 2
NS = 16
CHUNK = 128
NCH = 392
EPT = NCH * CHUNK
EPAD = NS * EPT
K = 4
NG = NCH // K
NPAD = 50048
ROWS_PT = NPAD // NS
ZROWS = 136
NZ = ROWS_PT // ZROWS
DEGW = 16

_mesh = plsc.VectorSubcoreMesh(core_axis_name="c", subcore_axis_name="s")
_sc_params = pltpu.CompilerParams(use_tc_tiling_on_sc=False)


def _seg_body(h_hbm, src_hbm, dst_hbm, zeros_hbm, out_hbm,
              acc, idx_src, idx_dst, rows, zbuf, sem_g, sem_s):
    c = lax.axis_index("c")
    s = lax.axis_index("s")
    base_row = s * ROWS_PT

    pltpu.sync_copy(zeros_hbm, zbuf)

    @pl.loop(0, NZ)
    def _zero(i):
        pltpu.sync_copy(zbuf, acc.at[pl.ds(base_row + i * ZROWS, ZROWS), :])

    plsc.subcore_barrier()

    @pl.loop(0, NG)
    def _group(g):
        row0 = s * NCH + g * K
        pltpu.sync_copy(src_hbm.at[c, pl.ds(row0, K), :], idx_src)
        pltpu.sync_copy(dst_hbm.at[c, pl.ds(row0, K), :], idx_dst)
        gathers = [
            pltpu.async_copy(h_hbm.at[idx_src.at[j]], rows.at[j], sem_g)
            for j in range(K)
        ]
        for d in gathers:
            d.wait()
        scatters = [
            pltpu.async_copy(rows.at[j], acc.at[idx_dst.at[j]], sem_s, add=True)
            for j in range(K)
        ]
        for d in scatters:
            d.wait()

    plsc.subcore_barrier()

    @pl.loop(0, NZ)
    def _out(i):
        r0 = base_row + i * ZROWS
        pltpu.sync_copy(acc.at[pl.ds(r0, ZROWS), :], zbuf)
        pltpu.sync_copy(zbuf, out_hbm.at[c, pl.ds(r0, ZROWS), :])


_seg_call = pl.kernel(
    _seg_body,
    out_type=jax.ShapeDtypeStruct((NC, NPAD, H), jnp.float32),
    mesh=_mesh,
    scratch_types=[
        pltpu.VMEM_SHARED((NPAD, H), jnp.float32),
        pltpu.VMEM((K, CHUNK), jnp.int32),
        pltpu.VMEM((K, CHUNK), jnp.int32),
        pltpu.VMEM((K, CHUNK, H), jnp.float32),
        pltpu.VMEM((ZROWS, H), jnp.float32),
        pltpu.SemaphoreType.DMA,
        pltpu.SemaphoreType.DMA,
    ],
    compiler_params=_sc_params,
)


def _deg_body(dst_hbm, ones_hbm, zeros_hbm, out_hbm,
              acc, idx_dst, ones_v, zbuf, sem_s):
    c = lax.axis_index("c")
    s = lax.axis_index("s")
    base_row = s * ROWS_PT

    pltpu.sync_copy(zeros_hbm, zbuf)
    pltpu.sync_copy(ones_hbm, ones_v)

    @pl.loop(0, NZ)
    def _zero(i):
        pltpu.sync_copy(zbuf, acc.at[pl.ds(base_row + i * ZROWS, ZROWS), :])

    plsc.subcore_barrier()

    @pl.loop(0, NG)
    def _group(g):
        row0 = s * NCH + g * K
        pltpu.sync_copy(dst_hbm.at[c, pl.ds(row0, K), :], idx_dst)
        scatters = [
            pltpu.async_copy(ones_v, acc.at[idx_dst.at[j]], sem_s, add=True)
            for j in range(K)
        ]
        for d in scatters:
            d.wait()

    plsc.subcore_barrier()

    @pl.loop(0, NZ)
    def _out(i):
        r0 = base_row + i * ZROWS
        pltpu.sync_copy(acc.at[pl.ds(r0, ZROWS), :], zbuf)
        pltpu.sync_copy(zbuf, out_hbm.at[c, pl.ds(r0, ZROWS), :])


_deg_call = pl.kernel(
    _deg_body,
    out_type=jax.ShapeDtypeStruct((NC, NPAD, DEGW), jnp.float32),
    mesh=_mesh,
    scratch_types=[
        pltpu.VMEM_SHARED((NPAD, DEGW), jnp.float32),
        pltpu.VMEM((K, CHUNK), jnp.int32),
        pltpu.VMEM((CHUNK, DEGW), jnp.float32),
        pltpu.VMEM((ZROWS, DEGW), jnp.float32),
        pltpu.SemaphoreType.DMA,
    ],
    compiler_params=_sc_params,
)


BN = 1088


def _combine_body(seg_ref, deg_ref, w0_ref, w1_ref, b_ref, out_ref):
    a0 = seg_ref[0]
    a1 = seg_ref[1]
    r0 = 1.0 / jnp.maximum(deg_ref[0, :, 0:1], 1.0)
    r1 = 1.0 / jnp.maximum(deg_ref[1, :, 0:1], 1.0)
    z = (
        jnp.dot(a0 * r0, w0_ref[...], preferred_element_type=jnp.float32)
        + jnp.dot(a1 * r1, w1_ref[...], preferred_element_type=jnp.float32)
        + b_ref[...]
    )
    out_ref[...] = jnp.where(z >= 0.0, z, NEG_SLOPE * z)


_combine_call = pl.pallas_call(
    _combine_body,
    grid=(NPAD // BN,),
    in_specs=[
        pl.BlockSpec((NC, BN, H), lambda i: (0, i, 0)),
        pl.BlockSpec((NC, BN, DEGW), lambda i: (0, i, 0)),
        pl.BlockSpec((H, H), lambda i: (0, 0)),
        pl.BlockSpec((H, H), lambda i: (0, 0)),
        pl.BlockSpec((1, H), lambda i: (0, 0)),
    ],
    out_specs=pl.BlockSpec((BN, H), lambda i: (i, 0)),
    out_shape=jax.ShapeDtypeStruct((NPAD, H), jnp.float32),
)


def _pad_edges(row, fill):
    return jnp.concatenate(
        [row, jnp.full((EPAD - E,), fill, jnp.int32)]).reshape(NS * NCH, CHUNK)


@jax.jit
def kernel(x, edge_index0, edge_index1, W0, W1, b):
    src = jnp.stack([_pad_edges(edge_index0[0], 0),
                     _pad_edges(edge_index1[0], 0)])
    dst = jnp.stack([_pad_edges(edge_index0[1], N),
                     _pad_edges(edge_index1[1], N)])
    zeros_h = jnp.zeros((ZROWS, H), jnp.float32)
    zeros_d = jnp.zeros((ZROWS, DEGW), jnp.float32)
    ones_d = jnp.ones((CHUNK, DEGW), jnp.float32)
    b2 = b.reshape(1, H)

    deg = _deg_call(dst, ones_d, zeros_d)
    h = x
    for _ in range(2):
        seg = _seg_call(h, src, dst, zeros_h)
        h = _combine_call(seg, deg, W0, W1, b2)
    return h[:N]
